# Initial kernel scaffold; baseline (speedup 1.0000x reference)
import jax
import jax.numpy as jnp
from jax.experimental import pallas as pl


def kernel(x, edge_index, W, b):
    raise NotImplementedError("write your pallas kernel here")


# speedup vs baseline: 5.2983x; 5.2983x over previous
import functools

import jax
import jax.numpy as jnp
from jax import lax
from jax.experimental import pallas as pl
from jax.experimental.pallas import tpu as pltpu
from jax.experimental.pallas import tpu_sc as plsc

C = 256
N = 10000
K = 16

_INFO = plsc.get_sparse_core_info()
NW = _INFO.num_cores * _INFO.num_subcores
NPAD = 10240
PER_W = NPAD // NW
P = 2
CH = PER_W // P
ROWS = 2 * P * K
IDXW = PER_W * K


def _tc_tables(x_flat, W, b2):
    NB = 10
    BN = NPAD // NB

    def body(x_ref, w_ref, b_ref, u_ref, v_ref):
        xb = x_ref[...]
        w1 = w_ref[:, :C]
        w2 = w_ref[:, C:]
        dn = (((0,), (1,)), ((), ()))
        u = lax.dot_general(xb, w1 - w2, dn, preferred_element_type=jnp.float32)
        v = lax.dot_general(xb, w2, dn, preferred_element_type=jnp.float32)
        u_ref[...] = u + b_ref[0, :][None, :]
        v_ref[...] = v

    return pl.pallas_call(
        body,
        grid=(NB,),
        in_specs=[
            pl.BlockSpec((C, BN), lambda i: (0, i)),
            pl.BlockSpec((C, 2 * C), lambda i: (0, 0)),
            pl.BlockSpec((1, C), lambda i: (0, 0)),
        ],
        out_specs=[
            pl.BlockSpec((BN, C), lambda i: (i, 0)),
            pl.BlockSpec((BN, C), lambda i: (i, 0)),
        ],
        out_shape=[
            jax.ShapeDtypeStruct((NPAD, C), jnp.float32),
            jax.ShapeDtypeStruct((NPAD, C), jnp.float32),
        ],
    )(x_flat, W, b2)


def _sc_gather_max(U, V, iU, iV):
    mesh = plsc.VectorSubcoreMesh(core_axis_name="c", subcore_axis_name="s")

    @functools.partial(
        pl.kernel,
        mesh=mesh,
        out_type=jax.ShapeDtypeStruct((NPAD, C), jnp.float32),
        scratch_types=[
            pltpu.VMEM((IDXW + ROWS,), jnp.int32),
            pltpu.VMEM((IDXW + ROWS,), jnp.int32),
            pltpu.VMEM((2 * ROWS, C), jnp.float32),
            pltpu.VMEM((2 * P, C), jnp.float32),
            pltpu.SemaphoreType.DMA,
            pltpu.SemaphoreType.DMA,
            pltpu.SemaphoreType.DMA,
            pltpu.SemaphoreType.DMA,
        ],
    )
    def body(iU_hbm, iV_hbm, u_hbm, v_hbm, out_hbm,
             iU_v, iV_v, rows_v, out_v, gs0, gs1, os0, os1):
        wid = lax.axis_index("s") * _INFO.num_cores + lax.axis_index("c")
        base = wid * PER_W
        gsem = (gs0, gs1)
        osem = (os0, os1)

        pltpu.sync_copy(iU_hbm.at[pl.ds(base * K, IDXW + ROWS)], iU_v)
        pltpu.sync_copy(iV_hbm.at[pl.ds(base * K, IDXW + ROWS)], iV_v)

        def start(c, s):
            half = P * K
            pltpu.async_copy(
                u_hbm.at[iU_v.at[pl.ds(c * half, half)]],
                rows_v.at[pl.ds(s * ROWS, half)], gsem[s])
            pltpu.async_copy(
                v_hbm.at[iV_v.at[pl.ds(c * half, half)]],
                rows_v.at[pl.ds(s * ROWS + half, half)], gsem[s])

        def wait_gather(s):
            pltpu.make_async_copy(
                u_hbm.at[pl.ds(0, ROWS)], rows_v.at[pl.ds(s * ROWS, ROWS)],
                gsem[s]).wait()

        def wait_out(s):
            pltpu.make_async_copy(
                out_hbm.at[pl.ds(0, P)], out_v.at[pl.ds(s * P, P)],
                osem[s]).wait()

        start(0, 0)
        start(1, 1)

        def step(c2, _):
            for s in (0, 1):
                c = c2 * 2 + s
                wait_gather(s)

                @pl.when(c2 >= 1)
                def _():
                    wait_out(s)

                for p in range(P):
                    ur = s * ROWS + p * K
                    vr = s * ROWS + P * K + p * K
                    for ch in range(C // 16):
                        lane = pl.ds(ch * 16, 16)
                        m = rows_v[ur, lane] + rows_v[vr, lane]
                        for k in range(1, K):
                            m = jnp.maximum(
                                m, rows_v[ur + k, lane] + rows_v[vr + k, lane])
                        out_v[s * P + p, lane] = jnp.maximum(m, 0.0)

                pltpu.async_copy(
                    out_v.at[pl.ds(s * P, P)],
                    out_hbm.at[pl.ds(base + c * P, P)], osem[s])
                start(c + 2, s)
            return 0

        lax.fori_loop(0, CH // 2, step, 0)
        for s in (0, 1):
            wait_gather(s)
            wait_out(s)

    return body(iU, iV, U, V)


def kernel(x, edge_index, W, b):
    x_flat = jnp.pad(x.reshape(C, N), ((0, 0), (0, NPAD - N)))
    b2 = b.reshape(1, C)
    U, V = _tc_tables(x_flat, W, b2)

    pad = NPAD * K + ROWS - N * K
    iU = jnp.pad(edge_index[1].reshape(N * K), (0, pad))
    iV = jnp.pad(edge_index[0].reshape(N * K), (0, pad))

    out = _sc_gather_max(U, V, iU, iV)
    return out[:N].T.reshape(1, C, N, 1)


# speedup vs baseline: 7.3724x; 1.3915x over previous
import functools

import jax
import jax.numpy as jnp
from jax import lax
from jax.experimental import pallas as pl
from jax.experimental.pallas import tpu as pltpu
from jax.experimental.pallas import tpu_sc as plsc

C = 256
N = 10000
K = 16

_INFO = plsc.get_sparse_core_info()
NW = _INFO.num_cores * _INFO.num_subcores
NPAD = 10240
PER_W = NPAD // NW
P = 2
CH = PER_W // P
ROWS = 2 * P * K
SLOTS = 4
IDXW = CH * ROWS
LANES = 16


def _tc_tables(x_flat, W, b2):
    NB = 10
    BN = NPAD // NB

    def body(x_ref, w_ref, b_ref, t_ref):
        g = pl.program_id(0)
        xb = x_ref[...]
        w1 = w_ref[:, :C]
        w2 = w_ref[:, C:]
        weff = jnp.where(g == 0, w1 - w2, w2)
        beff = jnp.where(g == 0, b_ref[0, :], 0.0)
        dn = (((0,), (1,)), ((), ()))
        t = lax.dot_general(xb, weff, dn, preferred_element_type=jnp.float32)
        t_ref[...] = t + beff[None, :]

    return pl.pallas_call(
        body,
        grid=(2, NB),
        in_specs=[
            pl.BlockSpec((C, BN), lambda g, i: (0, i)),
            pl.BlockSpec((C, 2 * C), lambda g, i: (0, 0)),
            pl.BlockSpec((1, C), lambda g, i: (0, 0)),
        ],
        out_specs=pl.BlockSpec((BN, C), lambda g, i: (g * NB + i, 0)),
        out_shape=jax.ShapeDtypeStruct((2 * NPAD, C), jnp.float32),
    )(x_flat, W, b2)


def _sc_gather_max(T, idx):
    mesh = plsc.VectorSubcoreMesh(core_axis_name="c", subcore_axis_name="s")

    @functools.partial(
        pl.kernel,
        mesh=mesh,
        out_type=jax.ShapeDtypeStruct((NPAD, C), jnp.float32),
        scratch_types=[
            pltpu.VMEM((IDXW + SLOTS * ROWS,), jnp.int32),
            pltpu.VMEM((SLOTS * ROWS, C), jnp.float32),
            pltpu.VMEM((SLOTS * P, C), jnp.float32),
            pltpu.SemaphoreType.DMA,
            pltpu.SemaphoreType.DMA,
            pltpu.SemaphoreType.DMA,
            pltpu.SemaphoreType.DMA,
            pltpu.SemaphoreType.DMA,
            pltpu.SemaphoreType.DMA,
            pltpu.SemaphoreType.DMA,
            pltpu.SemaphoreType.DMA,
        ],
    )
    def body(idx_hbm, t_hbm, out_hbm, idx_v, rows_v, out_v,
             gs0, gs1, gs2, gs3, os0, os1, os2, os3):
        wid = lax.axis_index("s") * _INFO.num_cores + lax.axis_index("c")
        base = wid * PER_W
        gsem = (gs0, gs1, gs2, gs3)
        osem = (os0, os1, os2, os3)

        pltpu.sync_copy(idx_hbm.at[pl.ds(base * 2 * K, IDXW + SLOTS * ROWS)],
                        idx_v)

        def start(c, s):
            pltpu.async_copy(
                t_hbm.at[idx_v.at[pl.ds(c * ROWS, ROWS)]],
                rows_v.at[pl.ds(s * ROWS, ROWS)], gsem[s])

        def wait_gather(s):
            pltpu.make_async_copy(
                t_hbm.at[pl.ds(0, ROWS)], rows_v.at[pl.ds(s * ROWS, ROWS)],
                gsem[s]).wait()

        def wait_out(s):
            pltpu.make_async_copy(
                out_hbm.at[pl.ds(0, P)], out_v.at[pl.ds(s * P, P)],
                osem[s]).wait()

        for s in range(SLOTS):
            start(s, s)

        def step(c4, _):
            for s in range(SLOTS):
                c = c4 * SLOTS + s
                wait_gather(s)

                @pl.when(c4 >= 1)
                def _():
                    wait_out(s)

                def lane_step(ch, _):
                    lane = pl.ds(ch * LANES, LANES)
                    for p in range(P):
                        ur = s * ROWS + p * K
                        vr = s * ROWS + P * K + p * K
                        m = rows_v[ur, lane] + rows_v[vr, lane]
                        for k in range(1, K):
                            m = jnp.maximum(
                                m, rows_v[ur + k, lane] + rows_v[vr + k, lane])
                        out_v[s * P + p, lane] = jnp.maximum(m, 0.0)
                    return 0

                lax.fori_loop(0, C // LANES, lane_step, 0)

                pltpu.async_copy(
                    out_v.at[pl.ds(s * P, P)],
                    out_hbm.at[pl.ds(base + c * P, P)], osem[s])
                start(c + SLOTS, s)
            return 0

        lax.fori_loop(0, CH // SLOTS, step, 0)
        for s in range(SLOTS):
            wait_gather(s)
            wait_out(s)

    return body(idx, T)


def kernel(x, edge_index, W, b):
    x_flat = jnp.pad(x.reshape(C, N), ((0, 0), (0, NPAD - N)))
    b2 = b.reshape(1, C)
    T = _tc_tables(x_flat, W, b2)

    pad = NPAD * K - N * K
    iU = jnp.pad(edge_index[1].reshape(N * K), (0, pad)).reshape(NPAD // P,
                                                                 P * K)
    iV = jnp.pad(edge_index[0].reshape(N * K), (0, pad)).reshape(NPAD // P,
                                                                 P * K) + NPAD
    idx = jnp.concatenate([iU, iV], axis=1).reshape(-1)
    idx = jnp.pad(idx, (0, SLOTS * ROWS))

    out = _sc_gather_max(T, idx)
    return out[:N].T.reshape(1, C, N, 1)
